# baseline (speedup 1.0000x reference)
import jax
import jax.numpy as jnp
from jax.experimental import pallas as pl
from jax.experimental.pallas import tpu as pltpu

B, N, D = 4096, 512, 16
VEL_W, SUR_W = 0.6, 0.4
BR = 32
STEPS = B // BR
RWS = BR * N
SBR = 256
SSTEPS = B // SBR


def _salience_body(zt_ref, ztm_ref, p_ref, out_ref, idx_ref,
                   acc_ref, pt_ref, ny_ref):
    i = pl.program_id(0)
    zt = zt_ref[...]
    out_ref[...] = zt.reshape(BR, N, D)
    ztm = ztm_ref[...]

    ztT = zt.T
    ztmT = ztm.T

    @pl.when(i == 0)
    def _prep():
        pT = p_ref[...].T
        pt_ref[...] = jnp.tile(pT, (1, BR))
        ny_ref[...] = jnp.sqrt(
            jnp.sum(pT * pT, axis=0, keepdims=True))

    ptT = pt_ref[...]

    diffT = ztT - ztmT
    vel2 = jnp.sum(diffT * diffT, axis=0, keepdims=True)
    zdot = jnp.sum(ztT * ptT, axis=0, keepdims=True)
    nx2 = jnp.sum(ztT * ztT, axis=0, keepdims=True)

    vel2 = vel2.reshape(BR, N)
    zdot = zdot.reshape(BR, N)
    nx2 = nx2.reshape(BR, N)

    vel = jnp.sqrt(vel2)
    nx = jnp.sqrt(nx2)
    ny = ny_ref[...]
    cos = zdot / jnp.maximum(nx * ny, 1e-8)
    surprise = jnp.clip(1.0 - cos, 0.0, 2.0) / 2.0
    sal = VEL_W * vel + SUR_W * surprise

    mn = jnp.min(sal, axis=-1, keepdims=True)
    mx = jnp.max(sal, axis=-1, keepdims=True)
    saln = (sal - mn) / (mx - mn + 1e-8)
    bsum = jnp.sum(saln, axis=0, keepdims=True)

    @pl.when(i == 0)
    def _init():
        acc_ref[...] = bsum

    @pl.when(i != 0)
    def _accum():
        acc_ref[...] = acc_ref[...] + bsum

    @pl.when(i == STEPS - 1)
    def _finish():
        acc = acc_ref[...]
        m = jnp.max(acc)
        eid = jax.lax.broadcasted_iota(jnp.int32, (1, N), 1)
        idx_ref[0, 0] = jnp.min(jnp.where(acc == m, eid, jnp.int32(2**30)))


def _scatter_body(idx_ref, mt_ref, y_any, o_any, src_ref, sem):
    del y_any
    n = idx_ref[0, 0]
    src_ref[...] = jnp.broadcast_to(mt_ref[...], (B, D))
    cp = pltpu.make_async_copy(src_ref, o_any.at[:, n, :], sem)
    cp.start()
    cp.wait()


def kernel(z_t, z_tm1, prior, mask_token):
    z2 = z_t.reshape(B * N, D)
    zm2 = z_tm1.reshape(B * N, D)

    out_copy, idx = pl.pallas_call(
        _salience_body,
        grid=(STEPS,),
        in_specs=[
            pl.BlockSpec((RWS, D), lambda i: (i, 0)),
            pl.BlockSpec((RWS, D), lambda i: (i, 0)),
            pl.BlockSpec((N, D), lambda i: (0, 0)),
        ],
        out_specs=[
            pl.BlockSpec((BR, N, D), lambda i: (i, 0, 0)),
            pl.BlockSpec(memory_space=pltpu.SMEM),
        ],
        out_shape=[
            jax.ShapeDtypeStruct((B, N, D), jnp.float32),
            jax.ShapeDtypeStruct((1, 1), jnp.int32),
        ],
        scratch_shapes=[
            pltpu.VMEM((1, N), jnp.float32),
            pltpu.VMEM((D, RWS), jnp.float32),
            pltpu.VMEM((1, N), jnp.float32),
        ],
    )(z2, zm2, prior)

    mt2 = mask_token.reshape(1, D)

    masked = pl.pallas_call(
        _scatter_body,
        in_specs=[
            pl.BlockSpec(memory_space=pltpu.SMEM),
            pl.BlockSpec((1, D), lambda: (0, 0)),
            pl.BlockSpec(memory_space=pl.ANY),
        ],
        out_specs=pl.BlockSpec(memory_space=pl.ANY),
        out_shape=jax.ShapeDtypeStruct((B, N, D), jnp.float32),
        scratch_shapes=[
            pltpu.VMEM((B, D), jnp.float32),
            pltpu.SemaphoreType.DMA,
        ],
        input_output_aliases={2: 0},
    )(idx, mt2, out_copy)

    return masked


# speedup vs baseline: 1.2762x; 1.2762x over previous
import jax
import jax.numpy as jnp
from jax.experimental import pallas as pl
from jax.experimental.pallas import tpu as pltpu

B, N, D = 4096, 512, 16
VEL_W, SUR_W = 0.6, 0.4
BR = 32
STEPS = B // BR
RWS = BR * N
SBR = 256
SSTEPS = B // SBR


def _salience_body(zt_ref, ztm_ref, p_ref, out_ref, idx_ref,
                   acc_ref, pt_ref, ny_ref):
    i = pl.program_id(0)
    zt = zt_ref[...]
    out_ref[...] = zt
    ztm = ztm_ref[...]

    ztT = zt.T
    ztmT = ztm.T

    @pl.when(i == 0)
    def _prep():
        pT = p_ref[...].T
        pt_ref[...] = jnp.tile(pT, (1, BR))
        ny_ref[...] = jnp.sqrt(
            jnp.sum(pT * pT, axis=0, keepdims=True))

    ptT = pt_ref[...]

    diffT = ztT - ztmT
    vel2 = jnp.sum(diffT * diffT, axis=0, keepdims=True)
    zdot = jnp.sum(ztT * ptT, axis=0, keepdims=True)
    nx2 = jnp.sum(ztT * ztT, axis=0, keepdims=True)

    vel2 = vel2.reshape(BR, N)
    zdot = zdot.reshape(BR, N)
    nx2 = nx2.reshape(BR, N)

    vel = jnp.sqrt(vel2)
    nx = jnp.sqrt(nx2)
    ny = ny_ref[...]
    cos = zdot / jnp.maximum(nx * ny, 1e-8)
    surprise = jnp.clip(1.0 - cos, 0.0, 2.0) / 2.0
    sal = VEL_W * vel + SUR_W * surprise

    mn = jnp.min(sal, axis=-1, keepdims=True)
    mx = jnp.max(sal, axis=-1, keepdims=True)
    saln = (sal - mn) / (mx - mn + 1e-8)
    bsum = jnp.sum(saln, axis=0, keepdims=True)

    @pl.when(i == 0)
    def _init():
        acc_ref[...] = bsum

    @pl.when(i != 0)
    def _accum():
        acc_ref[...] = acc_ref[...] + bsum

    @pl.when(i == STEPS - 1)
    def _finish():
        acc = acc_ref[...]
        m = jnp.max(acc)
        eid = jax.lax.broadcasted_iota(jnp.int32, (1, N), 1)
        idx_ref[0, 0] = jnp.min(jnp.where(acc == m, eid, jnp.int32(2**30)))


def _scatter_body(idx_ref, mt_ref, y_any, o_any, src_ref, sem):
    del y_any
    n = idx_ref[0, 0]
    src_ref[...] = jnp.broadcast_to(mt_ref[...], (B, D))
    o3 = o_any.reshape(B, N, D)
    cp = pltpu.make_async_copy(src_ref, o3.at[:, n, :], sem)
    cp.start()
    cp.wait()


def kernel(z_t, z_tm1, prior, mask_token):
    z2 = z_t.reshape(B * N, D)
    zm2 = z_tm1.reshape(B * N, D)

    out_copy, idx = pl.pallas_call(
        _salience_body,
        grid=(STEPS,),
        in_specs=[
            pl.BlockSpec((RWS, D), lambda i: (i, 0)),
            pl.BlockSpec((RWS, D), lambda i: (i, 0)),
            pl.BlockSpec((N, D), lambda i: (0, 0)),
        ],
        out_specs=[
            pl.BlockSpec((RWS, D), lambda i: (i, 0)),
            pl.BlockSpec(memory_space=pltpu.SMEM),
        ],
        out_shape=[
            jax.ShapeDtypeStruct((B * N, D), jnp.float32),
            jax.ShapeDtypeStruct((1, 1), jnp.int32),
        ],
        scratch_shapes=[
            pltpu.VMEM((1, N), jnp.float32),
            pltpu.VMEM((D, RWS), jnp.float32),
            pltpu.VMEM((1, N), jnp.float32),
        ],
    )(z2, zm2, prior)

    mt2 = mask_token.reshape(1, D)

    masked = pl.pallas_call(
        _scatter_body,
        in_specs=[
            pl.BlockSpec(memory_space=pltpu.SMEM),
            pl.BlockSpec((1, D), lambda: (0, 0)),
            pl.BlockSpec(memory_space=pl.ANY),
        ],
        out_specs=pl.BlockSpec(memory_space=pl.ANY),
        out_shape=jax.ShapeDtypeStruct((B * N, D), jnp.float32),
        scratch_shapes=[
            pltpu.VMEM((B, D), jnp.float32),
            pltpu.SemaphoreType.DMA,
        ],
        input_output_aliases={2: 0},
    )(idx, mt2, out_copy)

    return masked.reshape(B, N, D)
